# Initial kernel scaffold; baseline (speedup 1.0000x reference)
import jax
import jax.numpy as jnp
from jax.experimental import pallas as pl


def kernel(x, Wg, expert_bias, W1, b1, W2, b2, W3, b3, Ws1, bs1, Ws2, bs2, Ws3, bs3):
    raise NotImplementedError("write your pallas kernel here")


# speedup vs baseline: 1.2903x; 1.2903x over previous
import functools

import jax
import jax.numpy as jnp
from jax.experimental import pallas as pl
from jax.experimental.pallas import tpu as pltpu

T = 2048
DIM = 1024
DFF = 1408
E = 8
TOPK = 2
BM = 256
NEG = -3.0e38
NT = (((1,), (1,)), ((), ()))


def _routing_body(x_ref, wg_ref, bias_ref, tw_ref, idx_ref):
    x = x_ref[...]
    wg = wg_ref[...]
    logits = jax.lax.dot_general(x, wg, NT, preferred_element_type=jnp.float32)
    gw = jax.nn.sigmoid(logits)
    biased = logits + bias_ref[...]
    e_iota = jax.lax.broadcasted_iota(jnp.int32, (T, E), 1)
    m1 = jnp.max(biased, axis=1, keepdims=True)
    i1 = jnp.min(jnp.where(biased == m1, e_iota, E), axis=1, keepdims=True)
    masked = jnp.where(e_iota == i1, NEG, biased)
    m2 = jnp.max(masked, axis=1, keepdims=True)
    i2 = jnp.min(jnp.where(masked == m2, e_iota, E), axis=1, keepdims=True)
    w1 = jnp.sum(jnp.where(e_iota == i1, gw, 0.0), axis=1, keepdims=True)
    w2 = jnp.sum(jnp.where(e_iota == i2, gw, 0.0), axis=1, keepdims=True)
    s = w1 + w2
    tw_ref[...] = jnp.concatenate([w1 / s, w2 / s], axis=1)
    idx_ref[...] = jnp.concatenate([i1, i2], axis=1)


def _routing(x, Wg, expert_bias):
    return pl.pallas_call(
        _routing_body,
        out_shape=(
            jax.ShapeDtypeStruct((T, TOPK), jnp.float32),
            jax.ShapeDtypeStruct((T, TOPK), jnp.int32),
        ),
    )(x, Wg, expert_bias.reshape(1, E))


def _ffn_block(x, W1, b1, W2, b2, W3, b3):
    h1 = jax.lax.dot_general(x, W1, NT, preferred_element_type=jnp.float32) + b1
    h3 = jax.lax.dot_general(x, W3, NT, preferred_element_type=jnp.float32) + b3
    p = h1 * h3
    h = p * jax.nn.sigmoid(p)
    return jax.lax.dot_general(h, W2, NT, preferred_element_type=jnp.float32) + b2


def _dense_body(x_ref, w1_ref, b1_ref, w2_ref, b2_ref, w3_ref, b3_ref,
                tw_ref, idx_ref, y_ref, acc_ref):
    e = pl.program_id(0)
    b = pl.program_id(1)
    o = _ffn_block(x_ref[...], w1_ref[0], b1_ref[0], w2_ref[0], b2_ref[0],
                   w3_ref[0], b3_ref[0])
    w = jnp.sum(jnp.where(idx_ref[...] == e, tw_ref[...], 0.0), axis=1)
    contrib = o * w[:, None]
    rows = pl.ds(b * BM, BM)

    @pl.when(e == 0)
    def _():
        acc_ref[rows, :] = contrib

    @pl.when(e != 0)
    def _():
        acc_ref[rows, :] += contrib

    @pl.when(e == E - 1)
    def _():
        y_ref[...] = acc_ref[rows, :]


def _dense_moe(x, W1, b1, W2, b2, W3, b3, tw, idx):
    grid = (E, T // BM)
    emap = lambda e, b: (e, 0, 0)
    return pl.pallas_call(
        _dense_body,
        grid=grid,
        in_specs=[
            pl.BlockSpec((BM, DIM), lambda e, b: (b, 0)),
            pl.BlockSpec((1, DFF, DIM), emap),
            pl.BlockSpec((1, 1, DFF), emap),
            pl.BlockSpec((1, DIM, DFF), emap),
            pl.BlockSpec((1, 1, DIM), emap),
            pl.BlockSpec((1, DFF, DIM), emap),
            pl.BlockSpec((1, 1, DFF), emap),
            pl.BlockSpec((BM, TOPK), lambda e, b: (b, 0)),
            pl.BlockSpec((BM, TOPK), lambda e, b: (b, 0)),
        ],
        out_specs=pl.BlockSpec((BM, DIM), lambda e, b: (b, 0)),
        out_shape=jax.ShapeDtypeStruct((T, DIM), jnp.float32),
        scratch_shapes=[pltpu.VMEM((T, DIM), jnp.float32)],
    )(x, W1, b1.reshape(E, 1, DFF), W2, b2.reshape(E, 1, DIM),
      W3, b3.reshape(E, 1, DFF), tw, idx)


def _shared_body(x_ref, ws1_ref, bs1_ref, ws2_ref, bs2_ref, ws3_ref, bs3_ref,
                 ymoe_ref, y_ref):
    z = _ffn_block(x_ref[...], ws1_ref[...], bs1_ref[...], ws2_ref[...],
                   bs2_ref[...], ws3_ref[...], bs3_ref[...])
    y_ref[...] = z + ymoe_ref[...]


def _shared_ffn(x, Ws1, bs1, Ws2, bs2, Ws3, bs3, ymoe):
    zmap = lambda b: (0, 0)
    return pl.pallas_call(
        _shared_body,
        grid=(T // BM,),
        in_specs=[
            pl.BlockSpec((BM, DIM), lambda b: (b, 0)),
            pl.BlockSpec((DFF, DIM), zmap),
            pl.BlockSpec((1, DFF), zmap),
            pl.BlockSpec((DIM, DFF), zmap),
            pl.BlockSpec((1, DIM), zmap),
            pl.BlockSpec((DFF, DIM), zmap),
            pl.BlockSpec((1, DFF), zmap),
            pl.BlockSpec((BM, DIM), lambda b: (b, 0)),
        ],
        out_specs=pl.BlockSpec((BM, DIM), lambda b: (b, 0)),
        out_shape=jax.ShapeDtypeStruct((T, DIM), jnp.float32),
    )(x, Ws1, bs1.reshape(1, DFF), Ws2, bs2.reshape(1, DIM),
      Ws3, bs3.reshape(1, DFF), ymoe)


def kernel(x, Wg, expert_bias, W1, b1, W2, b2, W3, b3, Ws1, bs1, Ws2, bs2, Ws3, bs3):
    tw, idx = _routing(x, Wg, expert_bias)
    ymoe = _dense_moe(x, W1, b1, W2, b2, W3, b3, tw, idx)
    y = _shared_ffn(x, Ws1, bs1, Ws2, bs2, Ws3, bs3, ymoe)
    return (y, tw, idx)


# speedup vs baseline: 1.6048x; 1.2438x over previous
import functools

import jax
import jax.numpy as jnp
from jax import lax
from jax.experimental import pallas as pl
from jax.experimental.pallas import tpu as pltpu
from jax.experimental.pallas import tpu_sc as plsc

T = 2048
DIM = 1024
DFF = 1408
E = 8
TOPK = 2
NPAIR = T * TOPK
BM = 256
BG = 256
NBG = NPAIR // BG
G = NBG + E - 1
NEG = -3.0e38
NT = (((1,), (1,)), ((), ()))

NW = 32
GCH = 64
CCH = 32


def _routing_body(x_ref, wg_ref, bias_ref, tw_ref, idx_ref):
    x = x_ref[...]
    wg = wg_ref[...]
    logits = jax.lax.dot_general(x, wg, NT, preferred_element_type=jnp.float32)
    gw = jax.nn.sigmoid(logits)
    biased = logits + bias_ref[...]
    e_iota = jax.lax.broadcasted_iota(jnp.int32, (T, E), 1)
    m1 = jnp.max(biased, axis=1, keepdims=True)
    i1 = jnp.min(jnp.where(biased == m1, e_iota, E), axis=1, keepdims=True)
    masked = jnp.where(e_iota == i1, NEG, biased)
    m2 = jnp.max(masked, axis=1, keepdims=True)
    i2 = jnp.min(jnp.where(masked == m2, e_iota, E), axis=1, keepdims=True)
    w1 = jnp.sum(jnp.where(e_iota == i1, gw, 0.0), axis=1, keepdims=True)
    w2 = jnp.sum(jnp.where(e_iota == i2, gw, 0.0), axis=1, keepdims=True)
    s = w1 + w2
    tw_ref[...] = jnp.concatenate([w1 / s, w2 / s], axis=1)
    idx_ref[...] = jnp.concatenate([i1, i2], axis=1)


def _routing(x, Wg, expert_bias):
    return pl.pallas_call(
        _routing_body,
        out_shape=(
            jax.ShapeDtypeStruct((T, TOPK), jnp.float32),
            jax.ShapeDtypeStruct((T, TOPK), jnp.int32),
        ),
    )(x, Wg, expert_bias.reshape(1, E))


def _metadata(idx, tw):
    e_ids = idx.reshape(-1)
    order = jnp.argsort(e_ids, stable=True).astype(jnp.int32)
    sort_tok = (order // TOPK).astype(jnp.int32)
    inv = jnp.zeros((NPAIR,), jnp.int32).at[order].set(
        jnp.arange(NPAIR, dtype=jnp.int32))
    pos = inv.reshape(T, TOPK)
    w_sorted = tw.reshape(-1)[order]
    e_sorted = e_ids[order]
    counts = jnp.zeros((E,), jnp.int32).at[e_ids].add(1)
    ends = jnp.cumsum(counts)
    starts = ends - counts
    b_lo = starts // BG
    b_hi = jnp.maximum(ends - 1, 0) // BG
    nblk = jnp.where(counts > 0, b_hi - b_lo + 1, 0)
    cum = jnp.cumsum(nblk)
    g_actual = cum[-1]
    steps = jnp.arange(G, dtype=jnp.int32)
    e_step = jnp.minimum(
        jnp.searchsorted(cum, steps, side="right").astype(jnp.int32), E - 1)
    prev_cum = jnp.where(e_step > 0, cum[jnp.maximum(e_step - 1, 0)], 0)
    blk_step = b_lo[e_step] + (steps - prev_cum)
    valid = steps < g_actual
    last = jnp.maximum(g_actual - 1, 0)
    e_step = jnp.where(valid, e_step, e_step[last])
    blk_step = jnp.where(valid, blk_step, blk_step[last])
    prev_max = jnp.concatenate(
        [jnp.full((1,), -1, jnp.int32), lax.cummax(blk_step)[:-1]])
    first_visit = blk_step > prev_max
    meta = jnp.stack([e_step, blk_step, first_visit.astype(jnp.int32),
                      valid.astype(jnp.int32)])
    return meta, sort_tok, pos, w_sorted, e_sorted


def _sc_gather(tok, xsrc):
    nch = NPAIR // (NW * GCH)
    mesh = plsc.VectorSubcoreMesh(core_axis_name="c", subcore_axis_name="s")

    @functools.partial(
        pl.kernel, mesh=mesh,
        out_type=jax.ShapeDtypeStruct((NPAIR, DIM), jnp.float32),
        scratch_types=[
            pltpu.VMEM((GCH,), jnp.int32),
            pltpu.VMEM((GCH, DIM), jnp.float32),
            pltpu.SemaphoreType.DMA,
        ],
    )
    def k(tok_hbm, x_hbm, out_hbm, idx_v, rows_v, sem):
        wid = lax.axis_index("s") * 2 + lax.axis_index("c")
        for c in range(nch):
            base = wid * (GCH * nch) + c * GCH
            pltpu.sync_copy(tok_hbm.at[pl.ds(base, GCH)], idx_v)
            pltpu.async_copy(x_hbm.at[idx_v], rows_v, sem).wait()
            pltpu.sync_copy(rows_v, out_hbm.at[pl.ds(base, GCH)])

    return k(tok, xsrc)


def _ffn_block(x, W1, b1, W2, b2, W3, b3):
    h1 = jax.lax.dot_general(x, W1, NT, preferred_element_type=jnp.float32) + b1
    h3 = jax.lax.dot_general(x, W3, NT, preferred_element_type=jnp.float32) + b3
    p = h1 * h3
    h = p * jax.nn.sigmoid(p)
    return jax.lax.dot_general(h, W2, NT, preferred_element_type=jnp.float32) + b2


def _grouped_body(meta_ref, xs_ref, w1_ref, b1_ref, w2_ref, b2_ref,
                  w3_ref, b3_ref, wso_ref, eso_ref, ys_ref):
    i = pl.program_id(0)
    e = meta_ref[0, i]
    fv = meta_ref[2, i]
    valid = meta_ref[3, i]

    @pl.when(valid == 1)
    def _():
        o = _ffn_block(xs_ref[...], w1_ref[0], b1_ref[0], w2_ref[0],
                       b2_ref[0], w3_ref[0], b3_ref[0])
        w = jnp.where(eso_ref[0, 0, :] == e, wso_ref[0, 0, :], 0.0)
        contrib = o * w[:, None]

        @pl.when(fv == 1)
        def _():
            ys_ref[...] = contrib

        @pl.when(fv == 0)
        def _():
            ys_ref[...] += contrib


def _grouped_moe(meta, xs, W1, b1, W2, b2, W3, b3, w_sorted, e_sorted):
    grid_spec = pltpu.PrefetchScalarGridSpec(
        num_scalar_prefetch=1,
        grid=(G,),
        in_specs=[
            pl.BlockSpec((BG, DIM), lambda i, m: (m[1, i], 0)),
            pl.BlockSpec((1, DFF, DIM), lambda i, m: (m[0, i], 0, 0)),
            pl.BlockSpec((1, 1, DFF), lambda i, m: (m[0, i], 0, 0)),
            pl.BlockSpec((1, DIM, DFF), lambda i, m: (m[0, i], 0, 0)),
            pl.BlockSpec((1, 1, DIM), lambda i, m: (m[0, i], 0, 0)),
            pl.BlockSpec((1, DFF, DIM), lambda i, m: (m[0, i], 0, 0)),
            pl.BlockSpec((1, 1, DFF), lambda i, m: (m[0, i], 0, 0)),
            pl.BlockSpec((1, 1, BG), lambda i, m: (m[1, i], 0, 0)),
            pl.BlockSpec((1, 1, BG), lambda i, m: (m[1, i], 0, 0)),
        ],
        out_specs=pl.BlockSpec((BG, DIM), lambda i, m: (m[1, i], 0)),
    )
    return pl.pallas_call(
        _grouped_body,
        grid_spec=grid_spec,
        out_shape=jax.ShapeDtypeStruct((NPAIR, DIM), jnp.float32),
    )(meta, xs, W1, b1.reshape(E, 1, DFF), W2, b2.reshape(E, 1, DIM),
      W3, b3.reshape(E, 1, DFF),
      w_sorted.reshape(NBG, 1, BG), e_sorted.reshape(NBG, 1, BG))


def _sc_combine(pos0, pos1, ys):
    nch = T // (NW * CCH)
    mesh = plsc.VectorSubcoreMesh(core_axis_name="c", subcore_axis_name="s")

    @functools.partial(
        pl.kernel, mesh=mesh,
        out_type=jax.ShapeDtypeStruct((T, DIM), jnp.float32),
        scratch_types=[
            pltpu.VMEM((CCH,), jnp.int32),
            pltpu.VMEM((CCH,), jnp.int32),
            pltpu.VMEM((CCH, DIM), jnp.float32),
            pltpu.VMEM((CCH, DIM), jnp.float32),
            pltpu.SemaphoreType.DMA,
            pltpu.SemaphoreType.DMA,
        ],
    )
    def k(p0_hbm, p1_hbm, ys_hbm, out_hbm, i0_v, i1_v, r0_v, r1_v, s0, s1):
        wid = lax.axis_index("s") * 2 + lax.axis_index("c")
        for c in range(nch):
            base = wid * (CCH * nch) + c * CCH
            pltpu.sync_copy(p0_hbm.at[pl.ds(base, CCH)], i0_v)
            pltpu.sync_copy(p1_hbm.at[pl.ds(base, CCH)], i1_v)
            cp0 = pltpu.async_copy(ys_hbm.at[i0_v], r0_v, s0)
            cp1 = pltpu.async_copy(ys_hbm.at[i1_v], r1_v, s1)
            cp0.wait()
            cp1.wait()

            def row(r, carry):
                for cc in range(DIM // 16):
                    sl = pl.ds(cc * 16, 16)
                    r0_v[r, sl] += r1_v[r, sl]
                return carry

            lax.fori_loop(0, CCH, row, 0)
            pltpu.sync_copy(r0_v, out_hbm.at[pl.ds(base, CCH)])

    return k(pos0, pos1, ys)


def _shared_body(x_ref, ws1_ref, bs1_ref, ws2_ref, bs2_ref, ws3_ref, bs3_ref,
                 ymoe_ref, y_ref):
    z = _ffn_block(x_ref[...], ws1_ref[...], bs1_ref[...], ws2_ref[...],
                   bs2_ref[...], ws3_ref[...], bs3_ref[...])
    y_ref[...] = z + ymoe_ref[...]


def _shared_ffn(x, Ws1, bs1, Ws2, bs2, Ws3, bs3, ymoe):
    zmap = lambda b: (0, 0)
    return pl.pallas_call(
        _shared_body,
        grid=(T // BM,),
        in_specs=[
            pl.BlockSpec((BM, DIM), lambda b: (b, 0)),
            pl.BlockSpec((DFF, DIM), zmap),
            pl.BlockSpec((1, DFF), zmap),
            pl.BlockSpec((DIM, DFF), zmap),
            pl.BlockSpec((1, DIM), zmap),
            pl.BlockSpec((DFF, DIM), zmap),
            pl.BlockSpec((1, DFF), zmap),
            pl.BlockSpec((BM, DIM), lambda b: (b, 0)),
        ],
        out_specs=pl.BlockSpec((BM, DIM), lambda b: (b, 0)),
        out_shape=jax.ShapeDtypeStruct((T, DIM), jnp.float32),
    )(x, Ws1, bs1.reshape(1, DFF), Ws2, bs2.reshape(1, DIM),
      Ws3, bs3.reshape(1, DFF), ymoe)


def kernel(x, Wg, expert_bias, W1, b1, W2, b2, W3, b3, Ws1, bs1, Ws2, bs2, Ws3, bs3):
    tw, idx = _routing(x, Wg, expert_bias)
    meta, sort_tok, pos, w_sorted, e_sorted = _metadata(idx, tw)
    xs = _sc_gather(sort_tok, x)
    ys = _grouped_moe(meta, xs, W1, b1, W2, b2, W3, b3, w_sorted, e_sorted)
    ymoe = _sc_combine(pos[:, 0], pos[:, 1], ys)
    y = _shared_ffn(x, Ws1, bs1, Ws2, bs2, Ws3, bs3, ymoe)
    return (y, tw, idx)
